# baseline (speedup 1.0000x reference)
import jax
import jax.numpy as jnp
from jax import lax
from jax.experimental import pallas as pl
from jax.experimental.pallas import tpu as pltpu
from jax.experimental.pallas import tpu_sc as plsc

F = 26
V = 1000
D = 64
B = 1024
NC = 2
NS = 16
L = 16
NW = NC * NS
BW = B // NW
NPAIR = F * (F - 1) // 2
NSLOT = F * (F - 1)


def _sc_body(tab_hbm, xf_hbm, lin_hbm, out_hbm,
             x_v, lin_v, idx_v, vbuf, acc_v, out_v, sems):
    wid = lax.axis_index("s") * NC + lax.axis_index("c")
    base = wid * BW

    pltpu.sync_copy(xf_hbm.at[pl.ds(base * F, BW * F)], x_v)
    pltpu.sync_copy(lin_hbm, lin_v)

    iota = lax.iota(jnp.int32, L)
    zero = jnp.zeros((L,), jnp.float32)

    @pl.loop(0, BW, unroll=4)
    def _zero(r):
        for c in range(D // L):
            acc_v[r, pl.ds(c * L, L)] = zero

    rowbase = [(iota + h * L) * F for h in range(2)]

    first = [zero, zero]
    for i in range(F):
        xi = [plsc.load_gather(x_v, [rowbase[h] + i]) for h in range(2)]
        for h in range(2):
            first[h] = first[h] + plsc.load_gather(lin_v, [xi[h] + i * V])

        @pl.loop(0, F - 1, unroll=5)
        def _build(jj, i=i, xi=xi):
            j = jj + (jj >= i).astype(jnp.int32)
            s = i * (F - 1) + jj
            bofs = (i * F + j) * V
            idx_v[s, pl.ds(0, L)] = xi[0] + bofs
            idx_v[s, pl.ds(L, L)] = xi[1] + bofs

    def issue(ci, cj, par):
        s_ij = ci * (F - 1) + cj - 1
        s_ji = cj * (F - 1) + ci
        pltpu.async_copy(tab_hbm.at[idx_v.at[s_ij]], vbuf.at[par, 0],
                         sems.at[par])
        pltpu.async_copy(tab_hbm.at[idx_v.at[s_ji]], vbuf.at[par, 1],
                         sems.at[par])

    def wait_pair(par):
        for k in range(2):
            pltpu.make_async_copy(tab_hbm.at[idx_v.at[0]], vbuf.at[par, k],
                                  sems.at[par]).wait()

    def compute(par):
        @pl.loop(0, BW, unroll=4)
        def _rows(r):
            for c in range(D // (2 * L)):
                a = vbuf[par, 0, r, pl.ds(c * 2 * L, 2 * L)]
                b = vbuf[par, 1, r, pl.ds(c * 2 * L, 2 * L)]
                p0, p1 = plsc.unpack(a * b, format=plsc.PackFormat.INTERLEAVED)
                plsc.addupdate(acc_v.at[r, pl.ds(c * 2 * L, L)], p0)
                plsc.addupdate(acc_v.at[r, pl.ds(c * 2 * L + L, L)], p1)

    issue(jnp.int32(0), jnp.int32(1), 0)

    def advance(ci, cj):
        nj = cj + 1
        wrap = (nj >= F).astype(jnp.int32)
        return ci + wrap, jnp.where(wrap == 1, ci + 2, nj)

    @pl.loop(0, NPAIR - 1, step=2,
             init_carry=(jnp.int32(0), jnp.int32(2)))
    def _pairs(g, carry):
        ci, cj = carry
        for b in range(2):
            issue(ci, cj, 1 - b)
            wait_pair(b)
            compute(b)
            ci, cj = advance(ci, cj)
        return ci, cj

    wait_pair(0)
    compute(0)

    for h in range(2):
        tot = first[h]
        rows = iota + h * L
        for c in range(D):
            tot = tot + plsc.load_gather(acc_v, [rows, jnp.full((L,), c, jnp.int32)])
        sig = 1.0 / (1.0 + jnp.exp(-tot))
        out_v[pl.ds(h * L, L)] = sig
    pltpu.sync_copy(out_v, out_hbm.at[pl.ds(base, BW)])


@jax.jit
def _ffm_sc(tab, xf, lin):
    mesh = plsc.VectorSubcoreMesh(core_axis_name="c", subcore_axis_name="s",
                                  num_cores=NC, num_subcores=NS)
    return pl.kernel(
        _sc_body,
        out_type=jax.ShapeDtypeStruct((B,), jnp.float32),
        mesh=mesh,
        compiler_params=pltpu.CompilerParams(use_tc_tiling_on_sc=False,
                                             needs_layout_passes=False),
        scratch_types=[
            pltpu.VMEM((BW * F,), jnp.int32),
            pltpu.VMEM((F * V,), jnp.float32),
            pltpu.VMEM((NSLOT, BW), jnp.int32),
            pltpu.VMEM((2, 2, BW, D), jnp.bfloat16),
            pltpu.VMEM((BW, D), jnp.float32),
            pltpu.VMEM((BW,), jnp.float32),
            pltpu.SemaphoreType.DMA((2,)),
        ],
    )(tab, xf, lin)


def kernel(x, ffm_tables, lin_w, bias):
    tab = ffm_tables.astype(jnp.bfloat16).reshape(F * F * V, D)
    xf = x.reshape(B * F)
    lin = (lin_w + bias[0] / F).reshape(F * V)
    out = _ffm_sc(tab, xf, lin)
    return out.reshape(B, 1)


# speedup vs baseline: 1.3647x; 1.3647x over previous
import jax
import jax.numpy as jnp
from jax import lax
from jax.experimental import pallas as pl
from jax.experimental.pallas import tpu as pltpu
from jax.experimental.pallas import tpu_sc as plsc

F = 26
V = 1000
D = 64
B = 1024
NC = 2
NS = 16
L = 16
NW = NC * NS
BW = B // NW
NPAIR = F * (F - 1) // 2
NSLOT = F * (F - 1)


def _sc_body(tab_hbm, xf_hbm, lin_hbm, out_hbm,
             x_v, lin_v, idx_v, vbuf, acc_v, out_v, sems):
    wid = lax.axis_index("s") * NC + lax.axis_index("c")
    base = wid * BW

    pltpu.sync_copy(xf_hbm.at[pl.ds(base * F, BW * F)], x_v)
    pltpu.sync_copy(lin_hbm, lin_v)

    iota = lax.iota(jnp.int32, L)
    zero = jnp.zeros((L,), jnp.float32)

    @pl.loop(0, BW, unroll=4)
    def _zero(r):
        for c in range(D // L):
            acc_v[r, pl.ds(c * L, L)] = zero

    rowbase = [(iota + h * L) * F for h in range(2)]

    first = [zero, zero]
    for i in range(F):
        xi = [plsc.load_gather(x_v, [rowbase[h] + i]) for h in range(2)]
        for h in range(2):
            first[h] = first[h] + plsc.load_gather(lin_v, [xi[h] + i * V])

        @pl.loop(0, F - 1, unroll=5)
        def _build(jj, i=i, xi=xi):
            j = jj + (jj >= i).astype(jnp.int32)
            s = i * (F - 1) + jj
            bofs = (i * F + j) * V
            idx_v[s, pl.ds(0, L)] = xi[0] + bofs
            idx_v[s, pl.ds(L, L)] = xi[1] + bofs

    def issue(ci, cj, par):
        s_ij = ci * (F - 1) + cj - 1
        s_ji = cj * (F - 1) + ci
        pltpu.async_copy(tab_hbm.at[idx_v.at[s_ij]], vbuf.at[par, 0],
                         sems.at[par])
        pltpu.async_copy(tab_hbm.at[idx_v.at[s_ji]], vbuf.at[par, 1],
                         sems.at[par])

    def wait_pair(par):
        for k in range(2):
            pltpu.make_async_copy(tab_hbm.at[idx_v.at[0]], vbuf.at[par, k],
                                  sems.at[par]).wait()

    def compute(par):
        @pl.loop(0, BW, unroll=4)
        def _rows(r):
            for c in range(D // L):
                a = vbuf[par, 0, r, pl.ds(c * L, L)]
                b = vbuf[par, 1, r, pl.ds(c * L, L)]
                plsc.addupdate(acc_v.at[r, pl.ds(c * L, L)], a * b)

    issue(jnp.int32(0), jnp.int32(1), 0)
    issue(jnp.int32(0), jnp.int32(2), 1)

    def advance(ci, cj):
        nj = cj + 1
        wrap = (nj >= F).astype(jnp.int32)
        return ci + wrap, jnp.where(wrap == 1, ci + 2, nj)

    carry = (jnp.int32(0), jnp.int32(3))

    @pl.loop(0, NPAIR - 4, step=3, init_carry=carry)
    def _pairs(g, carry):
        ci, cj = carry
        for b in range(3):
            issue(ci, cj, (b + 2) % 3)
            wait_pair(b)
            compute(b)
            ci, cj = advance(ci, cj)
        return ci, cj

    ci, cj = _pairs
    issue(ci, cj, 2)
    wait_pair(0)
    compute(0)
    ci, cj = advance(ci, cj)
    issue(ci, cj, 0)
    wait_pair(1)
    compute(1)
    wait_pair(2)
    compute(2)
    wait_pair(0)
    compute(0)

    for h in range(2):
        tot = first[h]
        rows = iota + h * L
        for c in range(D):
            tot = tot + plsc.load_gather(acc_v, [rows, jnp.full((L,), c, jnp.int32)])
        sig = 1.0 / (1.0 + jnp.exp(-tot))
        out_v[pl.ds(h * L, L)] = sig
    pltpu.sync_copy(out_v, out_hbm.at[pl.ds(base, BW)])


@jax.jit
def _ffm_sc(tab, xf, lin):
    mesh = plsc.VectorSubcoreMesh(core_axis_name="c", subcore_axis_name="s",
                                  num_cores=NC, num_subcores=NS)
    return pl.kernel(
        _sc_body,
        out_type=jax.ShapeDtypeStruct((B,), jnp.float32),
        mesh=mesh,
        compiler_params=pltpu.CompilerParams(use_tc_tiling_on_sc=False,
                                             needs_layout_passes=False),
        scratch_types=[
            pltpu.VMEM((BW * F,), jnp.int32),
            pltpu.VMEM((F * V,), jnp.float32),
            pltpu.VMEM((NSLOT, BW), jnp.int32),
            pltpu.VMEM((3, 2, BW, D), jnp.float32),
            pltpu.VMEM((BW, D), jnp.float32),
            pltpu.VMEM((BW,), jnp.float32),
            pltpu.SemaphoreType.DMA((3,)),
        ],
    )(tab, xf, lin)


def kernel(x, ffm_tables, lin_w, bias):
    tab = ffm_tables.reshape(F * F * V, D)
    xf = x.reshape(B * F)
    lin = (lin_w + bias[0] / F).reshape(F * V)
    out = _ffm_sc(tab, xf, lin)
    return out.reshape(B, 1)


# speedup vs baseline: 1.4285x; 1.0467x over previous
import jax
import jax.numpy as jnp
from jax import lax
from jax.experimental import pallas as pl
from jax.experimental.pallas import tpu as pltpu
from jax.experimental.pallas import tpu_sc as plsc

F = 26
V = 1000
D = 64
B = 1024
NC = 2
NS = 16
L = 16
NW = NC * NS
BW = B // NW
NPAIR = F * (F - 1) // 2
NSLOT = F * (F - 1)


def _sc_body(tab_hbm, xf_hbm, lin_hbm, out_hbm,
             x_v, lin_v, idx_v, vbuf, acc_v, out_v, sems):
    wid = lax.axis_index("s") * NC + lax.axis_index("c")
    base = wid * BW

    pltpu.sync_copy(xf_hbm.at[pl.ds(base * F, BW * F)], x_v)
    pltpu.sync_copy(lin_hbm, lin_v)

    iota = lax.iota(jnp.int32, L)
    zero = jnp.zeros((L,), jnp.float32)

    @pl.loop(0, BW, unroll=4)
    def _zero(r):
        for c in range(D // L):
            acc_v[r, pl.ds(c * L, L)] = zero

    rowbase = [(iota + h * L) * F for h in range(2)]

    first = [zero, zero]
    for i in range(F):
        xi = [plsc.load_gather(x_v, [rowbase[h] + i]) for h in range(2)]
        for h in range(2):
            first[h] = first[h] + plsc.load_gather(lin_v, [xi[h] + i * V])

        @pl.loop(0, F - 1, unroll=5)
        def _build(jj, i=i, xi=xi):
            j = jj + (jj >= i).astype(jnp.int32)
            s = i * (F - 1) + jj
            bofs = (i * F + j) * V
            idx_v[s, pl.ds(0, L)] = xi[0] + bofs
            idx_v[s, pl.ds(L, L)] = xi[1] + bofs

    def issue(ci, cj, par):
        s_ij = ci * (F - 1) + cj - 1
        s_ji = cj * (F - 1) + ci
        pltpu.async_copy(tab_hbm.at[idx_v.at[s_ij]], vbuf.at[par, 0],
                         sems.at[par])
        pltpu.async_copy(tab_hbm.at[idx_v.at[s_ji]], vbuf.at[par, 1],
                         sems.at[par])

    def wait_pair(par):
        for k in range(2):
            pltpu.make_async_copy(tab_hbm.at[idx_v.at[0]], vbuf.at[par, k],
                                  sems.at[par]).wait()

    def compute(par):
        @pl.loop(0, BW, unroll=4)
        def _rows(r):
            for c in range(D // L):
                a = vbuf[par, 0, r, pl.ds(c * L, L)]
                b = vbuf[par, 1, r, pl.ds(c * L, L)]
                plsc.addupdate(acc_v.at[r, pl.ds(c * L, L)], a * b)

    issue(jnp.int32(0), jnp.int32(1), 0)
    issue(jnp.int32(0), jnp.int32(2), 1)
    issue(jnp.int32(0), jnp.int32(3), 2)

    def advance(ci, cj):
        nj = cj + 1
        wrap = (nj >= F).astype(jnp.int32)
        return ci + wrap, jnp.where(wrap == 1, ci + 2, nj)

    carry = (jnp.int32(0), jnp.int32(4))

    @pl.loop(0, NPAIR - 5, step=4, init_carry=carry)
    def _pairs(g, carry):
        ci, cj = carry
        for b in range(4):
            issue(ci, cj, (b + 3) % 4)
            wait_pair(b)
            compute(b)
            ci, cj = advance(ci, cj)
        return ci, cj

    ci, cj = _pairs
    issue(ci, cj, 3)
    wait_pair(0)
    compute(0)
    ci, cj = advance(ci, cj)
    issue(ci, cj, 0)
    wait_pair(1)
    compute(1)
    wait_pair(2)
    compute(2)
    wait_pair(3)
    compute(3)
    wait_pair(0)
    compute(0)

    for h in range(2):
        tot = first[h]
        rows = iota + h * L
        for c in range(D):
            tot = tot + plsc.load_gather(acc_v, [rows, jnp.full((L,), c, jnp.int32)])
        sig = 1.0 / (1.0 + jnp.exp(-tot))
        out_v[pl.ds(h * L, L)] = sig
    pltpu.sync_copy(out_v, out_hbm.at[pl.ds(base, BW)])


@jax.jit
def _ffm_sc(tab, xf, lin):
    mesh = plsc.VectorSubcoreMesh(core_axis_name="c", subcore_axis_name="s",
                                  num_cores=NC, num_subcores=NS)
    return pl.kernel(
        _sc_body,
        out_type=jax.ShapeDtypeStruct((B,), jnp.float32),
        mesh=mesh,
        compiler_params=pltpu.CompilerParams(use_tc_tiling_on_sc=False,
                                             needs_layout_passes=False),
        scratch_types=[
            pltpu.VMEM((BW * F,), jnp.int32),
            pltpu.VMEM((F * V,), jnp.float32),
            pltpu.VMEM((NSLOT, BW), jnp.int32),
            pltpu.VMEM((4, 2, BW, D), jnp.float32),
            pltpu.VMEM((BW, D), jnp.float32),
            pltpu.VMEM((BW,), jnp.float32),
            pltpu.SemaphoreType.DMA((4,)),
        ],
    )(tab, xf, lin)


def kernel(x, ffm_tables, lin_w, bias):
    tab = ffm_tables.reshape(F * F * V, D)
    xf = x.reshape(B * F)
    lin = (lin_w + bias[0] / F).reshape(F * V)
    out = _ffm_sc(tab, xf, lin)
    return out.reshape(B, 1)


# speedup vs baseline: 1.4648x; 1.0255x over previous
import jax
import jax.numpy as jnp
from jax import lax
from jax.experimental import pallas as pl
from jax.experimental.pallas import tpu as pltpu
from jax.experimental.pallas import tpu_sc as plsc

F = 26
V = 1000
D = 64
B = 1024
NC = 2
NS = 16
L = 16
NW = NC * NS
BW = B // NW
NPAIR = F * (F - 1) // 2
NSLOT = F * (F - 1)


def _sc_body(tab_hbm, xf_hbm, lin_hbm, out_hbm,
             x_v, lin_v, idx_v, vbuf, acc_v, out_v, sems):
    wid = lax.axis_index("s") * NC + lax.axis_index("c")
    base = wid * BW

    pltpu.sync_copy(xf_hbm.at[pl.ds(base * F, BW * F)], x_v)
    pltpu.sync_copy(lin_hbm, lin_v)

    iota = lax.iota(jnp.int32, L)
    zero = jnp.zeros((L,), jnp.float32)

    @pl.loop(0, BW, unroll=4)
    def _zero(r):
        for c in range(D // L):
            acc_v[r, pl.ds(c * L, L)] = zero

    rowbase = [(iota + h * L) * F for h in range(2)]

    first = [zero, zero]
    for i in range(F):
        xi = [plsc.load_gather(x_v, [rowbase[h] + i]) for h in range(2)]
        for h in range(2):
            first[h] = first[h] + plsc.load_gather(lin_v, [xi[h] + i * V])

        @pl.loop(0, F - 1, unroll=5)
        def _build(jj, i=i, xi=xi):
            j = jj + (jj >= i).astype(jnp.int32)
            s = i * (F - 1) + jj
            bofs = (i * F + j) * V
            idx_v[s, pl.ds(0, L)] = xi[0] + bofs
            idx_v[s, pl.ds(L, L)] = xi[1] + bofs

    def issue(ci, cj, par):
        s_ij = ci * (F - 1) + cj - 1
        s_ji = cj * (F - 1) + ci
        pltpu.async_copy(tab_hbm.at[idx_v.at[s_ij]], vbuf.at[par, 0],
                         sems.at[par])
        pltpu.async_copy(tab_hbm.at[idx_v.at[s_ji]], vbuf.at[par, 1],
                         sems.at[par])

    def wait_pair(par):
        for k in range(2):
            pltpu.make_async_copy(tab_hbm.at[idx_v.at[0]], vbuf.at[par, k],
                                  sems.at[par]).wait()

    def compute(par):
        @pl.loop(0, BW, unroll=4)
        def _rows(r):
            for c in range(D // L):
                a = vbuf[par, 0, r, pl.ds(c * L, L)]
                b = vbuf[par, 1, r, pl.ds(c * L, L)]
                plsc.addupdate(acc_v.at[r, pl.ds(c * L, L)], a * b)

    issue(jnp.int32(0), jnp.int32(1), 0)
    issue(jnp.int32(0), jnp.int32(2), 1)
    issue(jnp.int32(0), jnp.int32(3), 2)
    issue(jnp.int32(0), jnp.int32(4), 3)
    issue(jnp.int32(0), jnp.int32(5), 4)

    def advance(ci, cj):
        nj = cj + 1
        wrap = (nj >= F).astype(jnp.int32)
        return ci + wrap, jnp.where(wrap == 1, ci + 2, nj)

    carry = (jnp.int32(0), jnp.int32(6))

    @pl.loop(0, NPAIR - 7, step=6, init_carry=carry)
    def _pairs(g, carry):
        ci, cj = carry
        for b in range(6):
            issue(ci, cj, (b + 5) % 6)
            wait_pair(b)
            compute(b)
            ci, cj = advance(ci, cj)
        return ci, cj

    ci, cj = _pairs
    issue(ci, cj, 5)
    wait_pair(0)
    compute(0)
    ci, cj = advance(ci, cj)
    issue(ci, cj, 0)
    wait_pair(1)
    compute(1)
    wait_pair(2)
    compute(2)
    wait_pair(3)
    compute(3)
    wait_pair(4)
    compute(4)
    wait_pair(5)
    compute(5)
    wait_pair(0)
    compute(0)

    for h in range(2):
        tot = first[h]
        rows = iota + h * L
        for c in range(D):
            tot = tot + plsc.load_gather(acc_v, [rows, jnp.full((L,), c, jnp.int32)])
        sig = 1.0 / (1.0 + jnp.exp(-tot))
        out_v[pl.ds(h * L, L)] = sig
    pltpu.sync_copy(out_v, out_hbm.at[pl.ds(base, BW)])


@jax.jit
def _ffm_sc(tab, xf, lin):
    mesh = plsc.VectorSubcoreMesh(core_axis_name="c", subcore_axis_name="s",
                                  num_cores=NC, num_subcores=NS)
    return pl.kernel(
        _sc_body,
        out_type=jax.ShapeDtypeStruct((B,), jnp.float32),
        mesh=mesh,
        compiler_params=pltpu.CompilerParams(use_tc_tiling_on_sc=False,
                                             needs_layout_passes=False),
        scratch_types=[
            pltpu.VMEM((BW * F,), jnp.int32),
            pltpu.VMEM((F * V,), jnp.float32),
            pltpu.VMEM((NSLOT, BW), jnp.int32),
            pltpu.VMEM((6, 2, BW, D), jnp.float32),
            pltpu.VMEM((BW, D), jnp.float32),
            pltpu.VMEM((BW,), jnp.float32),
            pltpu.SemaphoreType.DMA((6,)),
        ],
    )(tab, xf, lin)


def kernel(x, ffm_tables, lin_w, bias):
    tab = ffm_tables.reshape(F * F * V, D)
    xf = x.reshape(B * F)
    lin = (lin_w + bias[0] / F).reshape(F * V)
    out = _ffm_sc(tab, xf, lin)
    return out.reshape(B, 1)
